# baseline (speedup 1.0000x reference)
import functools

import jax
import jax.numpy as jnp
from jax import lax
from jax.experimental import pallas as pl
from jax.experimental.pallas import tpu as pltpu
from jax.experimental.pallas import tpu_sc as plsc

_NC = 2
_NS = 16
_NW = _NC * _NS
_FEAT = 64
_BATCH = 16384
_CLS = 100000
_BLK = 2048
_NBLK = _BATCH // _BLK
_U = 32


def _sc_body(ct, xt, labels, d2_out, row_v, lab_v, x_v, acc_v, sem_r, sem_l, sem_x, sem_o):
    c = lax.axis_index("c")
    s = lax.axis_index("s")
    w = c * _NS + s

    pltpu.async_copy(labels, lab_v, sem_l).wait()

    out_cp = [None, None]
    for fi in range(2):
        f = w * 2 + fi
        cp_row = pltpu.async_copy(ct.at[f], row_v, sem_r)
        cp_x = [None, None]
        cp_x[0] = pltpu.async_copy(xt.at[f, pl.ds(0, _BLK)], x_v.at[0], sem_x)
        cp_row.wait()
        for b in range(_NBLK):
            cur = b % 2
            nxt = 1 - cur
            if b + 1 < _NBLK:
                cp_x[nxt] = pltpu.async_copy(
                    xt.at[f, pl.ds((b + 1) * _BLK, _BLK)], x_v.at[nxt], sem_x
                )
            cp_x[cur].wait()
            if out_cp[cur] is not None:
                out_cp[cur].wait()

            @plsc.parallel_loop(0, _BLK, 16, unroll=_U)
            def grp(o, b=b, cur=cur):
                lv = lab_v[pl.ds(b * _BLK + o, 16)]
                xv = x_v[cur, pl.ds(o, 16)]
                cv = plsc.load_gather(row_v, [lv])
                d = xv - cv
                acc_v[cur, pl.ds(o, 16)] = d * d
            out_cp[cur] = pltpu.async_copy(
                acc_v.at[cur], d2_out.at[f, pl.ds(b * _BLK, _BLK)], sem_o
            )
    for cp in out_cp:
        if cp is not None:
            cp.wait()


_sc_call = functools.partial(
    pl.kernel,
    out_type=jax.ShapeDtypeStruct((_FEAT, _BATCH), jnp.float32),
    mesh=plsc.VectorSubcoreMesh(
        core_axis_name="c", subcore_axis_name="s", num_cores=_NC, num_subcores=_NS
    ),
    compiler_params=pltpu.CompilerParams(needs_layout_passes=False),
    scratch_types=[
        pltpu.VMEM((_CLS,), jnp.float32),
        pltpu.VMEM((_BATCH,), jnp.int32),
        pltpu.VMEM((2, _BLK), jnp.float32),
        pltpu.VMEM((2, _BLK), jnp.float32),
        pltpu.SemaphoreType.DMA,
        pltpu.SemaphoreType.DMA,
        pltpu.SemaphoreType.DMA,
        pltpu.SemaphoreType.DMA,
    ],
)(_sc_body)


def _tc_body(d2_ref, out_ref):
    t = jnp.sum(d2_ref[...], axis=0)
    out_ref[...] = jnp.sum(jnp.sqrt(t)).reshape(1, 1)


_tc_call = pl.pallas_call(
    _tc_body, out_shape=jax.ShapeDtypeStruct((1, 1), jnp.float32)
)


def kernel(x, labels, centre):
    d2 = _sc_call(centre.T, x.T, labels.astype(jnp.int32))
    return _tc_call(d2)[0, 0]


# speedup vs baseline: 1.0958x; 1.0958x over previous
import functools

import jax
import jax.numpy as jnp
from jax import lax
from jax.experimental import pallas as pl
from jax.experimental.pallas import tpu as pltpu
from jax.experimental.pallas import tpu_sc as plsc

_NC = 2
_NS = 16
_NW = _NC * _NS
_FEAT = 64
_BATCH = 16384
_CLS = 100000
_BLK = 2048
_NBLK = _BATCH // _BLK
_U = 8


def _sc_body(ct, xt, labels, d2_out, row_v, lab_v, x_v, acc_v, sem_r, sem_l, sem_x, sem_o):
    c = lax.axis_index("c")
    s = lax.axis_index("s")
    w = c * _NS + s

    pltpu.async_copy(labels, lab_v, sem_l).wait()

    out_cp = [None, None]
    for fi in range(2):
        f = w * 2 + fi
        cp_row = pltpu.async_copy(ct.at[f], row_v, sem_r)
        cp_x = [None, None]
        cp_x[0] = pltpu.async_copy(xt.at[f, pl.ds(0, _BLK)], x_v.at[0], sem_x)
        cp_row.wait()
        for b in range(_NBLK):
            cur = b % 2
            nxt = 1 - cur
            if b + 1 < _NBLK:
                cp_x[nxt] = pltpu.async_copy(
                    xt.at[f, pl.ds((b + 1) * _BLK, _BLK)], x_v.at[nxt], sem_x
                )
            cp_x[cur].wait()
            if out_cp[cur] is not None:
                out_cp[cur].wait()

            @plsc.parallel_loop(0, _BLK, 16, unroll=_U)
            def grp(o, b=b, cur=cur):
                lv = lab_v[pl.ds(b * _BLK + o, 16)]
                xv = x_v[cur, pl.ds(o, 16)]
                cv = plsc.load_gather(row_v, [lv])
                d = xv - cv
                acc_v[cur, pl.ds(o, 16)] = d * d
            out_cp[cur] = pltpu.async_copy(
                acc_v.at[cur], d2_out.at[f, pl.ds(b * _BLK, _BLK)], sem_o
            )
    for cp in out_cp:
        if cp is not None:
            cp.wait()


_sc_call = functools.partial(
    pl.kernel,
    out_type=jax.ShapeDtypeStruct((_FEAT, _BATCH), jnp.float32),
    mesh=plsc.VectorSubcoreMesh(
        core_axis_name="c", subcore_axis_name="s", num_cores=_NC, num_subcores=_NS
    ),
    compiler_params=pltpu.CompilerParams(needs_layout_passes=False),
    scratch_types=[
        pltpu.VMEM((_CLS,), jnp.float32),
        pltpu.VMEM((_BATCH,), jnp.int32),
        pltpu.VMEM((2, _BLK), jnp.float32),
        pltpu.VMEM((2, _BLK), jnp.float32),
        pltpu.SemaphoreType.DMA,
        pltpu.SemaphoreType.DMA,
        pltpu.SemaphoreType.DMA,
        pltpu.SemaphoreType.DMA,
    ],
)(_sc_body)


def _tc_body(d2_ref, out_ref):
    t = jnp.sum(d2_ref[...], axis=0)
    out_ref[...] = jnp.sum(jnp.sqrt(t)).reshape(1, 1)


_tc_call = pl.pallas_call(
    _tc_body, out_shape=jax.ShapeDtypeStruct((1, 1), jnp.float32)
)


def kernel(x, labels, centre):
    d2 = _sc_call(centre.T, x.T, labels.astype(jnp.int32))
    return _tc_call(d2)[0, 0]


# speedup vs baseline: 1.1025x; 1.0062x over previous
import functools

import jax
import jax.numpy as jnp
from jax import lax
from jax.experimental import pallas as pl
from jax.experimental.pallas import tpu as pltpu
from jax.experimental.pallas import tpu_sc as plsc

_NC = 2
_NS = 16
_NW = _NC * _NS
_FEAT = 64
_BATCH = 16384
_CLS = 100000
_BLK = 2048
_NBLK = _BATCH // _BLK
_U = 4


def _sc_body(ct, xt, labels, d2_out, row_v, lab_v, x_v, acc_v, sem_r, sem_l, sem_x, sem_o):
    c = lax.axis_index("c")
    s = lax.axis_index("s")
    w = c * _NS + s

    pltpu.async_copy(labels, lab_v, sem_l).wait()

    out_cp = [None, None]
    for fi in range(2):
        f = w * 2 + fi
        cp_row = pltpu.async_copy(ct.at[f], row_v, sem_r)
        cp_x = [None, None]
        cp_x[0] = pltpu.async_copy(xt.at[f, pl.ds(0, _BLK)], x_v.at[0], sem_x)
        cp_row.wait()
        for b in range(_NBLK):
            cur = b % 2
            nxt = 1 - cur
            if b + 1 < _NBLK:
                cp_x[nxt] = pltpu.async_copy(
                    xt.at[f, pl.ds((b + 1) * _BLK, _BLK)], x_v.at[nxt], sem_x
                )
            cp_x[cur].wait()
            if out_cp[cur] is not None:
                out_cp[cur].wait()

            @plsc.parallel_loop(0, _BLK, 16, unroll=_U)
            def grp(o, b=b, cur=cur):
                lv = lab_v[pl.ds(b * _BLK + o, 16)]
                xv = x_v[cur, pl.ds(o, 16)]
                cv = plsc.load_gather(row_v, [lv])
                d = xv - cv
                acc_v[cur, pl.ds(o, 16)] = d * d
            out_cp[cur] = pltpu.async_copy(
                acc_v.at[cur], d2_out.at[f, pl.ds(b * _BLK, _BLK)], sem_o
            )
    for cp in out_cp:
        if cp is not None:
            cp.wait()


_sc_call = functools.partial(
    pl.kernel,
    out_type=jax.ShapeDtypeStruct((_FEAT, _BATCH), jnp.float32),
    mesh=plsc.VectorSubcoreMesh(
        core_axis_name="c", subcore_axis_name="s", num_cores=_NC, num_subcores=_NS
    ),
    compiler_params=pltpu.CompilerParams(needs_layout_passes=False),
    scratch_types=[
        pltpu.VMEM((_CLS,), jnp.float32),
        pltpu.VMEM((_BATCH,), jnp.int32),
        pltpu.VMEM((2, _BLK), jnp.float32),
        pltpu.VMEM((2, _BLK), jnp.float32),
        pltpu.SemaphoreType.DMA,
        pltpu.SemaphoreType.DMA,
        pltpu.SemaphoreType.DMA,
        pltpu.SemaphoreType.DMA,
    ],
)(_sc_body)


def _tc_body(d2_ref, out_ref):
    t = jnp.sum(d2_ref[...], axis=0)
    out_ref[...] = jnp.sum(jnp.sqrt(t)).reshape(1, 1)


_tc_call = pl.pallas_call(
    _tc_body, out_shape=jax.ShapeDtypeStruct((1, 1), jnp.float32)
)


def kernel(x, labels, centre):
    d2 = _sc_call(centre.T, x.T, labels.astype(jnp.int32))
    return _tc_call(d2)[0, 0]
